# baseline (speedup 1.0000x reference)
import functools

import jax
import jax.numpy as jnp
from jax import lax
from jax.experimental import pallas as pl
from jax.experimental.pallas import tpu as pltpu
from jax.experimental.pallas import tpu_sc as plsc

N = 10000
D = 128
NC = 2
NS = 16
NW = NC * NS
CH = 128
E = 320000
CHUNKS = 80
E_PAD = NW * CHUNKS * CH
NP = 10112
RPS = NP // NS
NPD = 10240
RPD = NPD // NS

_mesh = lambda: plsc.VectorSubcoreMesh(core_axis_name="c", subcore_axis_name="s")


def _deg_body(dst_hbm, z_hbm, out_hbm, idx_v, ones_v, acc):
    c = lax.axis_index("c")
    s = lax.axis_index("s")
    w = c * NS + s
    pltpu.sync_copy(z_hbm.at[pl.ds(s * RPD, RPD)], acc.at[pl.ds(s * RPD, RPD)])
    pltpu.sync_copy(dst_hbm.at[w], idx_v)
    for k in range(CH // 16):
        ones_v[pl.ds(k * 16, 16)] = jnp.ones((16,), jnp.float32)
    plsc.subcore_barrier()

    def body(j, carry):
        pltpu.sync_copy(ones_v, acc.at[idx_v.at[j]], add=True)
        return carry

    lax.fori_loop(0, CHUNKS, body, 0)
    plsc.subcore_barrier()
    pltpu.sync_copy(acc.at[pl.ds(s * RPD, RPD)], out_hbm.at[c, pl.ds(s * RPD, RPD)])


def _sc_degree(dst3, z1d):
    kern = pl.kernel(
        _deg_body,
        out_type=jax.ShapeDtypeStruct((NC, NPD), jnp.float32),
        mesh=_mesh(),
        scratch_types=[
            pltpu.VMEM((CHUNKS, CH), jnp.int32),
            pltpu.VMEM((CH,), jnp.float32),
            pltpu.VMEM_SHARED((NPD,), jnp.float32),
        ],
    )
    return kern(dst3, z1d)


def _spmm_body(xh_hbm, src_hbm, dst_hbm, z_hbm, out_hbm,
               src_v, dst_v, gb0, gb1,
               sg0, sg1, ss0, ss1, ss2, ss3, sd0, sd1, sd2, sd3, sw0, sw1,
               acc):
    c = lax.axis_index("c")
    s = lax.axis_index("s")
    w = c * NS + s
    GB = (gb0, gb1)
    SG = (sg0, sg1)
    SS = (ss0, ss1, ss2, ss3)
    SD = (sd0, sd1, sd2, sd3)
    SW = (sw0, sw1)

    pltpu.sync_copy(z_hbm.at[pl.ds(s * RPS, RPS)], acc.at[pl.ds(s * RPS, RPS)])
    plsc.subcore_barrier()

    pltpu.sync_copy(src_hbm.at[w, 0], src_v.at[0])
    pltpu.async_copy(xh_hbm.at[src_v.at[0]], gb0, sg0)
    for k in (1, 2):
        pltpu.async_copy(src_hbm.at[w, k], src_v.at[k], SS[k])
    for k in (0, 1, 2):
        pltpu.async_copy(dst_hbm.at[w, k], dst_v.at[k], SD[k])

    def body(i, carry):
        for u in range(4):
            j = 4 * i + u
            p, q = u % 2, 1 - u % 2
            kn, kp = (u + 1) % 4, (u + 3) % 4
            pltpu.make_async_copy(xh_hbm.at[src_v.at[u]], GB[p], SG[p]).wait()

            @pl.when(j + 1 < CHUNKS)
            def _():
                pltpu.make_async_copy(src_hbm.at[w, 0], src_v.at[kn], SS[kn]).wait()

                @pl.when(j >= 1)
                def _():
                    pltpu.make_async_copy(xh_hbm.at[src_v.at[0]], GB[q], SW[q]).wait()

                pltpu.async_copy(xh_hbm.at[src_v.at[kn]], GB[q], SG[q])

            @pl.when(j + 3 < CHUNKS)
            def _():
                pltpu.async_copy(src_hbm.at[w, j + 3], src_v.at[kp], SS[kp])

            pltpu.make_async_copy(dst_hbm.at[w, 0], dst_v.at[u], SD[u]).wait()
            pltpu.async_copy(GB[p], acc.at[dst_v.at[u]], SW[p], add=True)

            @pl.when(j + 3 < CHUNKS)
            def _():
                pltpu.async_copy(dst_hbm.at[w, j + 3], dst_v.at[kp], SD[kp])

        return carry

    lax.fori_loop(0, CHUNKS // 4, body, 0)
    pltpu.make_async_copy(xh_hbm.at[src_v.at[0]], gb0, sw0).wait()
    pltpu.make_async_copy(xh_hbm.at[src_v.at[1]], gb1, sw1).wait()
    plsc.subcore_barrier()
    pltpu.sync_copy(acc.at[pl.ds(s * RPS, RPS)], out_hbm.at[c, pl.ds(s * RPS, RPS)])


def _sc_spmm(xh, src3, dst3, z2d):
    kern = pl.kernel(
        _spmm_body,
        out_type=jax.ShapeDtypeStruct((NC, NP, D), jnp.float32),
        mesh=_mesh(),
        scratch_types=[
            pltpu.VMEM((4, CH), jnp.int32),
            pltpu.VMEM((4, CH), jnp.int32),
            pltpu.VMEM((CH, D), jnp.float32),
            pltpu.VMEM((CH, D), jnp.float32),
        ] + [pltpu.SemaphoreType.DMA] * 12 + [
            pltpu.VMEM_SHARED((NP, D), jnp.float32),
        ],
    )
    return kern(xh, src3, dst3, z2d)


_BLK = 1000


def _t0_body(degp_ref, x_ref, dinv_ref, xh_ref):
    deg = degp_ref[0] + degp_ref[1] + 1.0
    dinv = lax.rsqrt(deg)
    dinv_ref[...] = dinv
    xh_ref[...] = x_ref[...] * dinv


def _tc_t0(deg_parts, x):
    return pl.pallas_call(
        _t0_body,
        grid=(N // _BLK,),
        in_specs=[
            pl.BlockSpec((NC, _BLK, 1), lambda i: (0, i, 0)),
            pl.BlockSpec((_BLK, D), lambda i: (i, 0)),
        ],
        out_specs=[
            pl.BlockSpec((_BLK, 1), lambda i: (i, 0)),
            pl.BlockSpec((_BLK, D), lambda i: (i, 0)),
        ],
        out_shape=[
            jax.ShapeDtypeStruct((N, 1), jnp.float32),
            jax.ShapeDtypeStruct((N, D), jnp.float32),
        ],
    )(deg_parts, x)


def _layer_body(sp_ref, xh_ref, dinv_ref, w_ref, b_ref, p_ref, xhn_ref):
    dinv = dinv_ref[...]
    p = dinv * (sp_ref[0] + sp_ref[1] + xh_ref[...])
    p_ref[...] = p
    h = jnp.maximum(jnp.dot(p, w_ref[...],
                            preferred_element_type=jnp.float32) + b_ref[...], 0.0)
    xhn_ref[...] = dinv * h


def _tc_layer(s_parts, xh, dinv, W, b):
    return pl.pallas_call(
        _layer_body,
        grid=(N // _BLK,),
        in_specs=[
            pl.BlockSpec((NC, _BLK, D), lambda i: (0, i, 0)),
            pl.BlockSpec((_BLK, D), lambda i: (i, 0)),
            pl.BlockSpec((_BLK, 1), lambda i: (i, 0)),
            pl.BlockSpec((D, D), lambda i: (0, 0)),
            pl.BlockSpec((D,), lambda i: (0,)),
        ],
        out_specs=[
            pl.BlockSpec((_BLK, D), lambda i: (i, 0)),
            pl.BlockSpec((_BLK, D), lambda i: (i, 0)),
        ],
        out_shape=[
            jax.ShapeDtypeStruct((N, D), jnp.float32),
            jax.ShapeDtypeStruct((N, D), jnp.float32),
        ],
    )(s_parts, xh, dinv, W, b)


def _final_body(sp_ref, xh_ref, dinv_ref, p1_ref, p2_ref, wm_ref, bm_ref, out_ref):
    p3 = dinv_ref[...] * (sp_ref[0] + sp_ref[1] + xh_ref[...])
    acc = jnp.dot(p1_ref[...], wm_ref[0:D], preferred_element_type=jnp.float32)
    acc += jnp.dot(p2_ref[...], wm_ref[D:2 * D], preferred_element_type=jnp.float32)
    acc += jnp.dot(p3, wm_ref[2 * D:3 * D], preferred_element_type=jnp.float32)
    out_ref[...] = acc + bm_ref[...]


def _tc_final(s_parts, xh, dinv, p1, p2, Wm, bm):
    return pl.pallas_call(
        _final_body,
        grid=(N // _BLK,),
        in_specs=[
            pl.BlockSpec((NC, _BLK, D), lambda i: (0, i, 0)),
            pl.BlockSpec((_BLK, D), lambda i: (i, 0)),
            pl.BlockSpec((_BLK, 1), lambda i: (i, 0)),
            pl.BlockSpec((_BLK, D), lambda i: (i, 0)),
            pl.BlockSpec((_BLK, D), lambda i: (i, 0)),
            pl.BlockSpec((3 * D, D), lambda i: (0, 0)),
            pl.BlockSpec((D,), lambda i: (0,)),
        ],
        out_specs=pl.BlockSpec((_BLK, D), lambda i: (i, 0)),
        out_shape=jax.ShapeDtypeStruct((N, D), jnp.float32),
    )(s_parts, xh, dinv, p1, p2, Wm, bm)


@jax.jit
def kernel(x, edge_index, W1, b1, W2, b2, W3, b3, Wm, bm):
    src = edge_index[0]
    dst = edge_index[1]
    pad = E_PAD - E
    pad_ids = jnp.arange(pad, dtype=jnp.int32)
    src_pad = jnp.concatenate([src, pad_ids % N])
    dst_pad = jnp.concatenate([dst, N + (pad_ids % (NP - N))])
    src3 = src_pad.reshape(NW, CHUNKS, CH)
    dst3 = dst_pad.reshape(NW, CHUNKS, CH)
    z1d = jnp.zeros((NPD,), jnp.float32)
    z2d = jnp.zeros((NP, D), jnp.float32)

    deg_parts = _sc_degree(dst3, z1d).reshape(NC, NPD, 1)
    dinv, xh0 = _tc_t0(deg_parts, x)

    s0 = _sc_spmm(xh0, src3, dst3, z2d)
    _, xh1 = _tc_layer(s0, xh0, dinv, W1, b1)
    s1 = _sc_spmm(xh1, src3, dst3, z2d)
    p1, xh2 = _tc_layer(s1, xh1, dinv, W2, b2)
    s2 = _sc_spmm(xh2, src3, dst3, z2d)
    p2, xh3 = _tc_layer(s2, xh2, dinv, W3, b3)
    s3 = _sc_spmm(xh3, src3, dst3, z2d)
    return _tc_final(s3, xh3, dinv, p1, p2, Wm, bm)


# speedup vs baseline: 1.0189x; 1.0189x over previous
import functools

import jax
import jax.numpy as jnp
from jax import lax
from jax.experimental import pallas as pl
from jax.experimental.pallas import tpu as pltpu
from jax.experimental.pallas import tpu_sc as plsc

N = 10000
D = 128
NC = 2
NS = 16
NW = NC * NS
CH = 128
E = 320000
CHUNKS = 80
E_PAD = NW * CHUNKS * CH
NP = 10112
RPS = NP // NS
NPD = 10240
RPD = NPD // NS

_mesh = lambda: plsc.VectorSubcoreMesh(core_axis_name="c", subcore_axis_name="s")


def _deg_body(dst_hbm, z_hbm, out_hbm, idx_v, ones_v, acc):
    c = lax.axis_index("c")
    s = lax.axis_index("s")
    w = c * NS + s
    pltpu.sync_copy(z_hbm.at[pl.ds(s * RPD, RPD)], acc.at[pl.ds(s * RPD, RPD)])
    pltpu.sync_copy(dst_hbm.at[w], idx_v)
    for k in range(CH // 16):
        ones_v[pl.ds(k * 16, 16)] = jnp.ones((16,), jnp.float32)
    plsc.subcore_barrier()

    def body(j, carry):
        pltpu.sync_copy(ones_v, acc.at[idx_v.at[j]], add=True)
        return carry

    lax.fori_loop(0, CHUNKS, body, 0)
    plsc.subcore_barrier()
    pltpu.sync_copy(acc.at[pl.ds(s * RPD, RPD)], out_hbm.at[c, pl.ds(s * RPD, RPD)])


def _sc_degree(dst3, z1d):
    kern = pl.kernel(
        _deg_body,
        out_type=jax.ShapeDtypeStruct((NC, NPD), jnp.float32),
        mesh=_mesh(),
        scratch_types=[
            pltpu.VMEM((CHUNKS, CH), jnp.int32),
            pltpu.VMEM((CH,), jnp.float32),
            pltpu.VMEM_SHARED((NPD,), jnp.float32),
        ],
    )
    return kern(dst3, z1d)


HC = CH // 2


def _spmm_body(xh_hbm, src_hbm, dst_hbm, z_hbm, out_hbm,
               src_v, dst_v, gb0, gb1,
               sga0, sga1, sgb0, sgb1,
               ss0, ss1, ss2, ss3, sd0, sd1, sd2, sd3,
               swa0, swa1, swb0, swb1,
               acc):
    c = lax.axis_index("c")
    s = lax.axis_index("s")
    w = c * NS + s
    GB = (gb0, gb1)
    SGA = (sga0, sga1)
    SGB = (sgb0, sgb1)
    SS = (ss0, ss1, ss2, ss3)
    SD = (sd0, sd1, sd2, sd3)
    SWA = (swa0, swa1)
    SWB = (swb0, swb1)

    pltpu.sync_copy(z_hbm.at[pl.ds(s * RPS, RPS)], acc.at[pl.ds(s * RPS, RPS)])
    plsc.subcore_barrier()

    pltpu.sync_copy(src_hbm.at[w, 0], src_v.at[0])
    pltpu.async_copy(xh_hbm.at[src_v.at[0, pl.ds(0, HC)]], gb0.at[pl.ds(0, HC)], sga0)
    pltpu.async_copy(xh_hbm.at[src_v.at[0, pl.ds(HC, HC)]], gb0.at[pl.ds(HC, HC)], sgb0)
    for k in (1, 2):
        pltpu.async_copy(src_hbm.at[w, k], src_v.at[k], SS[k])
    for k in (0, 1, 2):
        pltpu.async_copy(dst_hbm.at[w, k], dst_v.at[k], SD[k])

    def body(i, carry):
        for u in range(4):
            j = 4 * i + u
            p, q = u % 2, 1 - u % 2
            kn, kp = (u + 1) % 4, (u + 3) % 4
            pltpu.make_async_copy(
                xh_hbm.at[src_v.at[u, pl.ds(0, HC)]], GB[p].at[pl.ds(0, HC)], SGA[p]).wait()
            pltpu.make_async_copy(
                xh_hbm.at[src_v.at[u, pl.ds(HC, HC)]], GB[p].at[pl.ds(HC, HC)], SGB[p]).wait()

            @pl.when(j + 1 < CHUNKS)
            def _():
                pltpu.make_async_copy(src_hbm.at[w, 0], src_v.at[kn], SS[kn]).wait()

                @pl.when(j >= 1)
                def _():
                    pltpu.make_async_copy(
                        GB[q].at[pl.ds(0, HC)],
                        acc.at[dst_v.at[kn, pl.ds(0, HC)]], SWA[q]).wait()
                    pltpu.make_async_copy(
                        GB[q].at[pl.ds(HC, HC)],
                        acc.at[dst_v.at[kn, pl.ds(HC, HC)]], SWB[q]).wait()

                pltpu.async_copy(
                    xh_hbm.at[src_v.at[kn, pl.ds(0, HC)]], GB[q].at[pl.ds(0, HC)], SGA[q])
                pltpu.async_copy(
                    xh_hbm.at[src_v.at[kn, pl.ds(HC, HC)]], GB[q].at[pl.ds(HC, HC)], SGB[q])

            @pl.when(j + 3 < CHUNKS)
            def _():
                pltpu.async_copy(src_hbm.at[w, j + 3], src_v.at[kp], SS[kp])

            pltpu.make_async_copy(dst_hbm.at[w, 0], dst_v.at[u], SD[u]).wait()
            pltpu.async_copy(GB[p].at[pl.ds(0, HC)],
                             acc.at[dst_v.at[u, pl.ds(0, HC)]], SWA[p], add=True)
            pltpu.async_copy(GB[p].at[pl.ds(HC, HC)],
                             acc.at[dst_v.at[u, pl.ds(HC, HC)]], SWB[p], add=True)

            @pl.when(j + 3 < CHUNKS)
            def _():
                pltpu.async_copy(dst_hbm.at[w, j + 3], dst_v.at[kp], SD[kp])

        return carry

    lax.fori_loop(0, CHUNKS // 4, body, 0)
    pltpu.make_async_copy(gb0.at[pl.ds(0, HC)],
                          acc.at[dst_v.at[2, pl.ds(0, HC)]], swa0).wait()
    pltpu.make_async_copy(gb0.at[pl.ds(HC, HC)],
                          acc.at[dst_v.at[2, pl.ds(HC, HC)]], swb0).wait()
    pltpu.make_async_copy(gb1.at[pl.ds(0, HC)],
                          acc.at[dst_v.at[3, pl.ds(0, HC)]], swa1).wait()
    pltpu.make_async_copy(gb1.at[pl.ds(HC, HC)],
                          acc.at[dst_v.at[3, pl.ds(HC, HC)]], swb1).wait()
    plsc.subcore_barrier()
    pltpu.sync_copy(acc.at[pl.ds(s * RPS, RPS)], out_hbm.at[c, pl.ds(s * RPS, RPS)])


def _sc_spmm(xh, src3, dst3, z2d):
    kern = pl.kernel(
        _spmm_body,
        out_type=jax.ShapeDtypeStruct((NC, NP, D), jnp.float32),
        mesh=_mesh(),
        scratch_types=[
            pltpu.VMEM((4, CH), jnp.int32),
            pltpu.VMEM((4, CH), jnp.int32),
            pltpu.VMEM((CH, D), jnp.float32),
            pltpu.VMEM((CH, D), jnp.float32),
        ] + [pltpu.SemaphoreType.DMA] * 16 + [
            pltpu.VMEM_SHARED((NP, D), jnp.float32),
        ],
    )
    return kern(xh, src3, dst3, z2d)


_BLK = 1000


def _t0_body(degp_ref, x_ref, dinv_ref, xh_ref):
    deg = degp_ref[0] + degp_ref[1] + 1.0
    dinv = lax.rsqrt(deg)
    dinv_ref[...] = dinv
    xh_ref[...] = x_ref[...] * dinv


def _tc_t0(deg_parts, x):
    return pl.pallas_call(
        _t0_body,
        grid=(N // _BLK,),
        in_specs=[
            pl.BlockSpec((NC, _BLK, 1), lambda i: (0, i, 0)),
            pl.BlockSpec((_BLK, D), lambda i: (i, 0)),
        ],
        out_specs=[
            pl.BlockSpec((_BLK, 1), lambda i: (i, 0)),
            pl.BlockSpec((_BLK, D), lambda i: (i, 0)),
        ],
        out_shape=[
            jax.ShapeDtypeStruct((N, 1), jnp.float32),
            jax.ShapeDtypeStruct((N, D), jnp.float32),
        ],
    )(deg_parts, x)


def _layer_body(sp_ref, xh_ref, dinv_ref, w_ref, b_ref, p_ref, xhn_ref):
    dinv = dinv_ref[...]
    p = dinv * (sp_ref[0] + sp_ref[1] + xh_ref[...])
    p_ref[...] = p
    h = jnp.maximum(jnp.dot(p, w_ref[...],
                            preferred_element_type=jnp.float32) + b_ref[...], 0.0)
    xhn_ref[...] = dinv * h


def _tc_layer(s_parts, xh, dinv, W, b):
    return pl.pallas_call(
        _layer_body,
        grid=(N // _BLK,),
        in_specs=[
            pl.BlockSpec((NC, _BLK, D), lambda i: (0, i, 0)),
            pl.BlockSpec((_BLK, D), lambda i: (i, 0)),
            pl.BlockSpec((_BLK, 1), lambda i: (i, 0)),
            pl.BlockSpec((D, D), lambda i: (0, 0)),
            pl.BlockSpec((D,), lambda i: (0,)),
        ],
        out_specs=[
            pl.BlockSpec((_BLK, D), lambda i: (i, 0)),
            pl.BlockSpec((_BLK, D), lambda i: (i, 0)),
        ],
        out_shape=[
            jax.ShapeDtypeStruct((N, D), jnp.float32),
            jax.ShapeDtypeStruct((N, D), jnp.float32),
        ],
    )(s_parts, xh, dinv, W, b)


def _final_body(sp_ref, xh_ref, dinv_ref, p1_ref, p2_ref, wm_ref, bm_ref, out_ref):
    p3 = dinv_ref[...] * (sp_ref[0] + sp_ref[1] + xh_ref[...])
    acc = jnp.dot(p1_ref[...], wm_ref[0:D], preferred_element_type=jnp.float32)
    acc += jnp.dot(p2_ref[...], wm_ref[D:2 * D], preferred_element_type=jnp.float32)
    acc += jnp.dot(p3, wm_ref[2 * D:3 * D], preferred_element_type=jnp.float32)
    out_ref[...] = acc + bm_ref[...]


def _tc_final(s_parts, xh, dinv, p1, p2, Wm, bm):
    return pl.pallas_call(
        _final_body,
        grid=(N // _BLK,),
        in_specs=[
            pl.BlockSpec((NC, _BLK, D), lambda i: (0, i, 0)),
            pl.BlockSpec((_BLK, D), lambda i: (i, 0)),
            pl.BlockSpec((_BLK, 1), lambda i: (i, 0)),
            pl.BlockSpec((_BLK, D), lambda i: (i, 0)),
            pl.BlockSpec((_BLK, D), lambda i: (i, 0)),
            pl.BlockSpec((3 * D, D), lambda i: (0, 0)),
            pl.BlockSpec((D,), lambda i: (0,)),
        ],
        out_specs=pl.BlockSpec((_BLK, D), lambda i: (i, 0)),
        out_shape=jax.ShapeDtypeStruct((N, D), jnp.float32),
    )(s_parts, xh, dinv, p1, p2, Wm, bm)


@jax.jit
def kernel(x, edge_index, W1, b1, W2, b2, W3, b3, Wm, bm):
    src = edge_index[0]
    dst = edge_index[1]
    pad = E_PAD - E
    pad_ids = jnp.arange(pad, dtype=jnp.int32)
    src_pad = jnp.concatenate([src, pad_ids % N])
    dst_pad = jnp.concatenate([dst, N + (pad_ids % (NP - N))])
    src3 = src_pad.reshape(NW, CHUNKS, CH)
    dst3 = dst_pad.reshape(NW, CHUNKS, CH)
    z1d = jnp.zeros((NPD,), jnp.float32)
    z2d = jnp.zeros((NP, D), jnp.float32)

    deg_parts = _sc_degree(dst3, z1d).reshape(NC, NPD, 1)
    dinv, xh0 = _tc_t0(deg_parts, x)

    s0 = _sc_spmm(xh0, src3, dst3, z2d)
    _, xh1 = _tc_layer(s0, xh0, dinv, W1, b1)
    s1 = _sc_spmm(xh1, src3, dst3, z2d)
    p1, xh2 = _tc_layer(s1, xh1, dinv, W2, b2)
    s2 = _sc_spmm(xh2, src3, dst3, z2d)
    p2, xh3 = _tc_layer(s2, xh2, dinv, W3, b3)
    s3 = _sc_spmm(xh3, src3, dst3, z2d)
    return _tc_final(s3, xh3, dinv, p1, p2, Wm, bm)


# speedup vs baseline: 1.1781x; 1.1562x over previous
import functools

import jax
import jax.numpy as jnp
from jax import lax
from jax.experimental import pallas as pl
from jax.experimental.pallas import tpu as pltpu
from jax.experimental.pallas import tpu_sc as plsc

N = 10000
D = 128
NC = 2
NS = 16
NW = NC * NS
CH = 88
E = 320000
CHUNKS = 120
E_PAD = NW * CHUNKS * CH
CH_D = 128
CHUNKS_D = 80
E_PAD_D = NW * CHUNKS_D * CH_D
NP = 10112
RPS = NP // NS
NPD = 10240
RPD = NPD // NS

_mesh = lambda: plsc.VectorSubcoreMesh(core_axis_name="c", subcore_axis_name="s")


def _deg_body(dst_hbm, z_hbm, out_hbm, idx_v, ones_v, acc):
    c = lax.axis_index("c")
    s = lax.axis_index("s")
    w = c * NS + s
    pltpu.sync_copy(z_hbm.at[pl.ds(s * RPD, RPD)], acc.at[pl.ds(s * RPD, RPD)])
    pltpu.sync_copy(dst_hbm.at[w], idx_v)
    for k in range(CH_D // 16):
        ones_v[pl.ds(k * 16, 16)] = jnp.ones((16,), jnp.float32)
    plsc.subcore_barrier()

    def body(j, carry):
        pltpu.sync_copy(ones_v, acc.at[idx_v.at[j]], add=True)
        return carry

    lax.fori_loop(0, CHUNKS_D, body, 0)
    plsc.subcore_barrier()
    pltpu.sync_copy(acc.at[pl.ds(s * RPD, RPD)], out_hbm.at[c, pl.ds(s * RPD, RPD)])


def _sc_degree(dst3, z1d):
    kern = pl.kernel(
        _deg_body,
        out_type=jax.ShapeDtypeStruct((NC, NPD), jnp.float32),
        mesh=_mesh(),
        scratch_types=[
            pltpu.VMEM((CHUNKS_D, CH_D), jnp.int32),
            pltpu.VMEM((CH_D,), jnp.float32),
            pltpu.VMEM_SHARED((NPD,), jnp.float32),
        ],
    )
    return kern(dst3, z1d)


def _spmm_body(xh_hbm, src_hbm, dst_hbm, z_hbm, out_hbm,
               src_v, dst_v, gb0, gb1, gb2, gb3,
               sg0, sg1, sg2, sg3, sw0, sw1, sw2, sw3,
               ss0, ss1, ss2, ss3, ss4, ss5, ss6, ss7,
               sd0, sd1, sd2, sd3, sd4, sd5, sd6, sd7,
               acc):
    c = lax.axis_index("c")
    s = lax.axis_index("s")
    w = c * NS + s
    GB = (gb0, gb1, gb2, gb3)
    SG = (sg0, sg1, sg2, sg3)
    SW = (sw0, sw1, sw2, sw3)
    SS = (ss0, ss1, ss2, ss3, ss4, ss5, ss6, ss7)
    SD = (sd0, sd1, sd2, sd3, sd4, sd5, sd6, sd7)

    pltpu.sync_copy(z_hbm.at[pl.ds(s * RPS, RPS)], acc.at[pl.ds(s * RPS, RPS)])
    plsc.subcore_barrier()

    for k in (0, 1, 2):
        pltpu.sync_copy(src_hbm.at[w, k], src_v.at[k])
        pltpu.async_copy(xh_hbm.at[src_v.at[k]], GB[k], SG[k])
    for k in (3, 4, 5):
        pltpu.async_copy(src_hbm.at[w, k], src_v.at[k], SS[k])
    for k in (0, 1, 2, 3, 4, 5):
        pltpu.async_copy(dst_hbm.at[w, k], dst_v.at[k], SD[k])

    def body(i, carry):
        for u in range(8):
            j = 8 * i + u
            u4 = u % 4
            v4 = (u + 3) % 4
            n8 = (u + 3) % 8
            f8 = (u + 6) % 8
            pltpu.make_async_copy(xh_hbm.at[src_v.at[u]], GB[u4], SG[u4]).wait()
            pltpu.make_async_copy(dst_hbm.at[w, 0], dst_v.at[u], SD[u]).wait()
            pltpu.async_copy(GB[u4], acc.at[dst_v.at[u]], SW[u4], add=True)

            @pl.when(j + 3 < CHUNKS)
            def _():
                pltpu.make_async_copy(src_hbm.at[w, 0], src_v.at[n8], SS[n8]).wait()

                @pl.when(j >= 1)
                def _():
                    pltpu.make_async_copy(
                        GB[v4], acc.at[dst_v.at[n8]], SW[v4]).wait()

                pltpu.async_copy(xh_hbm.at[src_v.at[n8]], GB[v4], SG[v4])

            @pl.when(j + 6 < CHUNKS)
            def _():
                pltpu.async_copy(src_hbm.at[w, j + 6], src_v.at[f8], SS[f8])
                pltpu.async_copy(dst_hbm.at[w, j + 6], dst_v.at[f8], SD[f8])

        return carry

    lax.fori_loop(0, CHUNKS // 8, body, 0)
    for k in (0, 1, 2, 3):
        pltpu.make_async_copy(GB[k], acc.at[dst_v.at[4 + k]], SW[k]).wait()
    plsc.subcore_barrier()
    pltpu.sync_copy(acc.at[pl.ds(s * RPS, RPS)], out_hbm.at[c, pl.ds(s * RPS, RPS)])


def _sc_spmm(xh, src3, dst3, z2d):
    kern = pl.kernel(
        _spmm_body,
        out_type=jax.ShapeDtypeStruct((NC, NP, D), jnp.float32),
        mesh=_mesh(),
        scratch_types=[
            pltpu.VMEM((8, CH), jnp.int32),
            pltpu.VMEM((8, CH), jnp.int32),
            pltpu.VMEM((CH, D), jnp.float32),
            pltpu.VMEM((CH, D), jnp.float32),
            pltpu.VMEM((CH, D), jnp.float32),
            pltpu.VMEM((CH, D), jnp.float32),
        ] + [pltpu.SemaphoreType.DMA] * 24 + [
            pltpu.VMEM_SHARED((NP, D), jnp.float32),
        ],
    )
    return kern(xh, src3, dst3, z2d)


_BLK = 1000


def _t0_body(degp_ref, x_ref, dinv_ref, xh_ref):
    deg = degp_ref[0] + degp_ref[1] + 1.0
    dinv = lax.rsqrt(deg)
    dinv_ref[...] = dinv
    xh_ref[...] = x_ref[...] * dinv


def _tc_t0(deg_parts, x):
    return pl.pallas_call(
        _t0_body,
        grid=(N // _BLK,),
        in_specs=[
            pl.BlockSpec((NC, _BLK, 1), lambda i: (0, i, 0)),
            pl.BlockSpec((_BLK, D), lambda i: (i, 0)),
        ],
        out_specs=[
            pl.BlockSpec((_BLK, 1), lambda i: (i, 0)),
            pl.BlockSpec((_BLK, D), lambda i: (i, 0)),
        ],
        out_shape=[
            jax.ShapeDtypeStruct((N, 1), jnp.float32),
            jax.ShapeDtypeStruct((N, D), jnp.float32),
        ],
    )(deg_parts, x)


def _layer_body(sp_ref, xh_ref, dinv_ref, w_ref, b_ref, p_ref, xhn_ref):
    dinv = dinv_ref[...]
    p = dinv * (sp_ref[0] + sp_ref[1] + xh_ref[...])
    p_ref[...] = p
    h = jnp.maximum(jnp.dot(p, w_ref[...],
                            preferred_element_type=jnp.float32) + b_ref[...], 0.0)
    xhn_ref[...] = dinv * h


def _tc_layer(s_parts, xh, dinv, W, b):
    return pl.pallas_call(
        _layer_body,
        grid=(N // _BLK,),
        in_specs=[
            pl.BlockSpec((NC, _BLK, D), lambda i: (0, i, 0)),
            pl.BlockSpec((_BLK, D), lambda i: (i, 0)),
            pl.BlockSpec((_BLK, 1), lambda i: (i, 0)),
            pl.BlockSpec((D, D), lambda i: (0, 0)),
            pl.BlockSpec((D,), lambda i: (0,)),
        ],
        out_specs=[
            pl.BlockSpec((_BLK, D), lambda i: (i, 0)),
            pl.BlockSpec((_BLK, D), lambda i: (i, 0)),
        ],
        out_shape=[
            jax.ShapeDtypeStruct((N, D), jnp.float32),
            jax.ShapeDtypeStruct((N, D), jnp.float32),
        ],
    )(s_parts, xh, dinv, W, b)


def _final_body(sp_ref, xh_ref, dinv_ref, p1_ref, p2_ref, wm_ref, bm_ref, out_ref):
    p3 = dinv_ref[...] * (sp_ref[0] + sp_ref[1] + xh_ref[...])
    acc = jnp.dot(p1_ref[...], wm_ref[0:D], preferred_element_type=jnp.float32)
    acc += jnp.dot(p2_ref[...], wm_ref[D:2 * D], preferred_element_type=jnp.float32)
    acc += jnp.dot(p3, wm_ref[2 * D:3 * D], preferred_element_type=jnp.float32)
    out_ref[...] = acc + bm_ref[...]


def _tc_final(s_parts, xh, dinv, p1, p2, Wm, bm):
    return pl.pallas_call(
        _final_body,
        grid=(N // _BLK,),
        in_specs=[
            pl.BlockSpec((NC, _BLK, D), lambda i: (0, i, 0)),
            pl.BlockSpec((_BLK, D), lambda i: (i, 0)),
            pl.BlockSpec((_BLK, 1), lambda i: (i, 0)),
            pl.BlockSpec((_BLK, D), lambda i: (i, 0)),
            pl.BlockSpec((_BLK, D), lambda i: (i, 0)),
            pl.BlockSpec((3 * D, D), lambda i: (0, 0)),
            pl.BlockSpec((D,), lambda i: (0,)),
        ],
        out_specs=pl.BlockSpec((_BLK, D), lambda i: (i, 0)),
        out_shape=jax.ShapeDtypeStruct((N, D), jnp.float32),
    )(s_parts, xh, dinv, p1, p2, Wm, bm)


@jax.jit
def kernel(x, edge_index, W1, b1, W2, b2, W3, b3, Wm, bm):
    src = edge_index[0]
    dst = edge_index[1]
    pad = E_PAD - E
    pad_ids = jnp.arange(pad, dtype=jnp.int32)
    src_pad = jnp.concatenate([src, pad_ids % N])
    dst_pad = jnp.concatenate([dst, N + (pad_ids % (NP - N))])
    src3 = src_pad.reshape(NW, CHUNKS, CH)
    dst3 = dst_pad.reshape(NW, CHUNKS, CH)
    pad_d = E_PAD_D - E
    pad_ids_d = jnp.arange(pad_d, dtype=jnp.int32)
    dst3d = jnp.concatenate(
        [dst, N + (pad_ids_d % (NPD - N))]).reshape(NW, CHUNKS_D, CH_D)
    z1d = jnp.zeros((NPD,), jnp.float32)
    z2d = jnp.zeros((NP, D), jnp.float32)

    deg_parts = _sc_degree(dst3d, z1d).reshape(NC, NPD, 1)
    dinv, xh0 = _tc_t0(deg_parts, x)

    s0 = _sc_spmm(xh0, src3, dst3, z2d)
    _, xh1 = _tc_layer(s0, xh0, dinv, W1, b1)
    s1 = _sc_spmm(xh1, src3, dst3, z2d)
    p1, xh2 = _tc_layer(s1, xh1, dinv, W2, b2)
    s2 = _sc_spmm(xh2, src3, dst3, z2d)
    p2, xh3 = _tc_layer(s2, xh2, dinv, W3, b3)
    s3 = _sc_spmm(xh3, src3, dst3, z2d)
    return _tc_final(s3, xh3, dinv, p1, p2, Wm, bm)


# speedup vs baseline: 1.1974x; 1.0164x over previous
import functools

import jax
import jax.numpy as jnp
from jax import lax
from jax.experimental import pallas as pl
from jax.experimental.pallas import tpu as pltpu
from jax.experimental.pallas import tpu_sc as plsc

N = 10000
D = 128
NC = 2
NS = 16
NW = NC * NS
CH = 88
E = 320000
CHUNKS = 120
E_PAD = NW * CHUNKS * CH
CH_D = 128
CHUNKS_D = 80
E_PAD_D = NW * CHUNKS_D * CH_D
NP = 10112
RPS = NP // NS
NPD = 10240
RPD = NPD // NS

_mesh = lambda: plsc.VectorSubcoreMesh(core_axis_name="c", subcore_axis_name="s")


def _deg_body(dst_hbm, z_hbm, out_hbm, idx_v, ones_v, acc):
    c = lax.axis_index("c")
    s = lax.axis_index("s")
    w = c * NS + s
    pltpu.sync_copy(z_hbm.at[pl.ds(s * RPD, RPD)], acc.at[pl.ds(s * RPD, RPD)])
    pltpu.sync_copy(dst_hbm.at[w], idx_v)
    for k in range(CH_D // 16):
        ones_v[pl.ds(k * 16, 16)] = jnp.ones((16,), jnp.float32)
    plsc.subcore_barrier()

    def body(j, carry):
        pltpu.sync_copy(ones_v, acc.at[idx_v.at[j]], add=True)
        return carry

    lax.fori_loop(0, CHUNKS_D, body, 0)
    plsc.subcore_barrier()
    pltpu.sync_copy(acc.at[pl.ds(s * RPD, RPD)], out_hbm.at[c, pl.ds(s * RPD, RPD)])


def _sc_degree(dst3, z1d):
    kern = pl.kernel(
        _deg_body,
        out_type=jax.ShapeDtypeStruct((NC, NPD), jnp.float32),
        mesh=_mesh(),
        scratch_types=[
            pltpu.VMEM((CHUNKS_D, CH_D), jnp.int32),
            pltpu.VMEM((CH_D,), jnp.float32),
            pltpu.VMEM_SHARED((NPD,), jnp.float32),
        ],
    )
    return kern(dst3, z1d)


def _spmm_body(xh_hbm, src_hbm, dst_hbm, z_hbm, out_hbm,
               src_v, dst_v, gb0, gb1, gb2, gb3,
               sg0, sg1, sg2, sg3, sw0, sw1, sw2, sw3,
               ss0, ss1, ss2, ss3, ss4, ss5, ss6, ss7,
               sd0, sd1, sd2, sd3, sd4, sd5, sd6, sd7,
               acc):
    c = lax.axis_index("c")
    s = lax.axis_index("s")
    w = c * NS + s
    GB = (gb0, gb1, gb2, gb3)
    SG = (sg0, sg1, sg2, sg3)
    SW = (sw0, sw1, sw2, sw3)
    SS = (ss0, ss1, ss2, ss3, ss4, ss5, ss6, ss7)
    SD = (sd0, sd1, sd2, sd3, sd4, sd5, sd6, sd7)

    pltpu.sync_copy(z_hbm.at[pl.ds(s * RPS, RPS)], acc.at[pl.ds(s * RPS, RPS)])
    plsc.subcore_barrier()

    for k in (0, 1, 2):
        pltpu.sync_copy(src_hbm.at[w, k], src_v.at[k])
        pltpu.async_copy(xh_hbm.at[src_v.at[k]], GB[k], SG[k])
    for k in (3, 4, 5):
        pltpu.async_copy(src_hbm.at[w, k], src_v.at[k], SS[k])
    for k in (0, 1, 2, 3, 4, 5):
        pltpu.async_copy(dst_hbm.at[w, k], dst_v.at[k], SD[k])

    def body(i, carry):
        for u in range(8):
            j = 8 * i + u
            u4 = u % 4
            v4 = (u + 3) % 4
            n8 = (u + 3) % 8
            f8 = (u + 6) % 8
            pltpu.make_async_copy(xh_hbm.at[src_v.at[u]], GB[u4], SG[u4]).wait()
            pltpu.make_async_copy(dst_hbm.at[w, 0], dst_v.at[u], SD[u]).wait()
            pltpu.async_copy(GB[u4], acc.at[dst_v.at[u]], SW[u4], add=True)

            @pl.when(j + 3 < CHUNKS)
            def _():
                pltpu.make_async_copy(src_hbm.at[w, 0], src_v.at[n8], SS[n8]).wait()

                @pl.when(j >= 1)
                def _():
                    pltpu.make_async_copy(
                        GB[v4], acc.at[dst_v.at[n8]], SW[v4]).wait()

                pltpu.async_copy(xh_hbm.at[src_v.at[n8]], GB[v4], SG[v4])

            @pl.when(j + 6 < CHUNKS)
            def _():
                pltpu.async_copy(src_hbm.at[w, j + 6], src_v.at[f8], SS[f8])
                pltpu.async_copy(dst_hbm.at[w, j + 6], dst_v.at[f8], SD[f8])

        return carry

    lax.fori_loop(0, CHUNKS // 8, body, 0)
    for k in (0, 1, 2, 3):
        pltpu.make_async_copy(GB[k], acc.at[dst_v.at[4 + k]], SW[k]).wait()
    plsc.subcore_barrier()
    pltpu.sync_copy(acc.at[pl.ds(s * RPS, RPS)], out_hbm.at[c, pl.ds(s * RPS, RPS)])


def _sc_spmm(xh, src3, dst3, z2d):
    kern = pl.kernel(
        _spmm_body,
        out_type=jax.ShapeDtypeStruct((NC, NP, D), jnp.float32),
        mesh=_mesh(),
        scratch_types=[
            pltpu.VMEM((8, CH), jnp.int32),
            pltpu.VMEM((8, CH), jnp.int32),
            pltpu.VMEM((CH, D), jnp.float32),
            pltpu.VMEM((CH, D), jnp.float32),
            pltpu.VMEM((CH, D), jnp.float32),
            pltpu.VMEM((CH, D), jnp.float32),
        ] + [pltpu.SemaphoreType.DMA] * 24 + [
            pltpu.VMEM_SHARED((NP, D), jnp.float32),
        ],
    )
    return kern(xh, src3, dst3, z2d)


_BLK = 2000


def _t0_body(degp_ref, x_ref, dinv_ref, xh_ref):
    deg = degp_ref[0] + degp_ref[1] + 1.0
    dinv = lax.rsqrt(deg)
    dinv_ref[...] = dinv
    xh_ref[...] = x_ref[...] * dinv


def _tc_t0(deg_parts, x):
    return pl.pallas_call(
        _t0_body,
        grid=(N // _BLK,),
        in_specs=[
            pl.BlockSpec((NC, _BLK, 1), lambda i: (0, i, 0)),
            pl.BlockSpec((_BLK, D), lambda i: (i, 0)),
        ],
        out_specs=[
            pl.BlockSpec((_BLK, 1), lambda i: (i, 0)),
            pl.BlockSpec((_BLK, D), lambda i: (i, 0)),
        ],
        out_shape=[
            jax.ShapeDtypeStruct((N, 1), jnp.float32),
            jax.ShapeDtypeStruct((N, D), jnp.float32),
        ],
    )(deg_parts, x)


def _layer_body(sp_ref, xh_ref, dinv_ref, w_ref, b_ref, p_ref, xhn_ref):
    dinv = dinv_ref[...]
    p = dinv * (sp_ref[0] + sp_ref[1] + xh_ref[...])
    p_ref[...] = p
    h = jnp.maximum(jnp.dot(p, w_ref[...],
                            preferred_element_type=jnp.float32) + b_ref[...], 0.0)
    xhn_ref[...] = dinv * h


def _tc_layer(s_parts, xh, dinv, W, b):
    return pl.pallas_call(
        _layer_body,
        grid=(N // _BLK,),
        in_specs=[
            pl.BlockSpec((NC, _BLK, D), lambda i: (0, i, 0)),
            pl.BlockSpec((_BLK, D), lambda i: (i, 0)),
            pl.BlockSpec((_BLK, 1), lambda i: (i, 0)),
            pl.BlockSpec((D, D), lambda i: (0, 0)),
            pl.BlockSpec((D,), lambda i: (0,)),
        ],
        out_specs=[
            pl.BlockSpec((_BLK, D), lambda i: (i, 0)),
            pl.BlockSpec((_BLK, D), lambda i: (i, 0)),
        ],
        out_shape=[
            jax.ShapeDtypeStruct((N, D), jnp.float32),
            jax.ShapeDtypeStruct((N, D), jnp.float32),
        ],
    )(s_parts, xh, dinv, W, b)


def _final_body(sp_ref, xh_ref, dinv_ref, p1_ref, p2_ref, wm_ref, bm_ref, out_ref):
    p3 = dinv_ref[...] * (sp_ref[0] + sp_ref[1] + xh_ref[...])
    acc = jnp.dot(p1_ref[...], wm_ref[0:D], preferred_element_type=jnp.float32)
    acc += jnp.dot(p2_ref[...], wm_ref[D:2 * D], preferred_element_type=jnp.float32)
    acc += jnp.dot(p3, wm_ref[2 * D:3 * D], preferred_element_type=jnp.float32)
    out_ref[...] = acc + bm_ref[...]


def _tc_final(s_parts, xh, dinv, p1, p2, Wm, bm):
    return pl.pallas_call(
        _final_body,
        grid=(N // _BLK,),
        in_specs=[
            pl.BlockSpec((NC, _BLK, D), lambda i: (0, i, 0)),
            pl.BlockSpec((_BLK, D), lambda i: (i, 0)),
            pl.BlockSpec((_BLK, 1), lambda i: (i, 0)),
            pl.BlockSpec((_BLK, D), lambda i: (i, 0)),
            pl.BlockSpec((_BLK, D), lambda i: (i, 0)),
            pl.BlockSpec((3 * D, D), lambda i: (0, 0)),
            pl.BlockSpec((D,), lambda i: (0,)),
        ],
        out_specs=pl.BlockSpec((_BLK, D), lambda i: (i, 0)),
        out_shape=jax.ShapeDtypeStruct((N, D), jnp.float32),
    )(s_parts, xh, dinv, p1, p2, Wm, bm)


@jax.jit
def kernel(x, edge_index, W1, b1, W2, b2, W3, b3, Wm, bm):
    src = edge_index[0]
    dst = edge_index[1]
    pad = E_PAD - E
    pad_ids = jnp.arange(pad, dtype=jnp.int32)
    src_pad = jnp.concatenate([src, pad_ids % N])
    dst_pad = jnp.concatenate([dst, N + (pad_ids % (NP - N))])
    src3 = src_pad.reshape(NW, CHUNKS, CH)
    dst3 = dst_pad.reshape(NW, CHUNKS, CH)
    pad_d = E_PAD_D - E
    pad_ids_d = jnp.arange(pad_d, dtype=jnp.int32)
    dst3d = jnp.concatenate(
        [dst, N + (pad_ids_d % (NPD - N))]).reshape(NW, CHUNKS_D, CH_D)
    z1d = jnp.zeros((NPD,), jnp.float32)
    z2d = jnp.zeros((NP, D), jnp.float32)

    deg_parts = _sc_degree(dst3d, z1d).reshape(NC, NPD, 1)
    dinv, xh0 = _tc_t0(deg_parts, x)

    s0 = _sc_spmm(xh0, src3, dst3, z2d)
    _, xh1 = _tc_layer(s0, xh0, dinv, W1, b1)
    s1 = _sc_spmm(xh1, src3, dst3, z2d)
    p1, xh2 = _tc_layer(s1, xh1, dinv, W2, b2)
    s2 = _sc_spmm(xh2, src3, dst3, z2d)
    p2, xh3 = _tc_layer(s2, xh2, dinv, W3, b3)
    s3 = _sc_spmm(xh3, src3, dst3, z2d)
    return _tc_final(s3, xh3, dinv, p1, p2, Wm, bm)
